# Initial kernel scaffold; baseline (speedup 1.0000x reference)
import jax
import jax.numpy as jnp
from jax.experimental import pallas as pl


def kernel(input, E0, W0, E1, W1, E2, W2):
    raise NotImplementedError("write your pallas kernel here")


# speedup vs baseline: 8.0783x; 8.0783x over previous
import functools

import jax
import jax.numpy as jnp
from jax import lax
from jax.experimental import pallas as pl
from jax.experimental.pallas import tpu as pltpu
from jax.experimental.pallas import tpu_sc as plsc

_EMBED = 128
_CUT0 = 20000
_CUT1 = 60000
_V_TOTAL = 1_000_000
_BLK = 4000
_NBLK = _V_TOTAL // _BLK
_SEG1 = _CUT0 // _BLK
_SEG2 = _CUT1 // _BLK

_LANES = 128
_CH_G = 4


def _proj_body(e0, w0, e1, w1, e2, w2, out):
    i = pl.program_id(0)

    @pl.when(i < _SEG1)
    def _():
        out[...] = jnp.dot(e0[...], w0[...], preferred_element_type=jnp.float32)

    @pl.when(jnp.logical_and(i >= _SEG1, i < _SEG2))
    def _():
        out[...] = jnp.dot(e1[...], w1[...], preferred_element_type=jnp.float32)

    @pl.when(i >= _SEG2)
    def _():
        out[...] = jnp.dot(e2[...], w2[...], preferred_element_type=jnp.float32)


def _build_projected_table(E0, W0, E1, W1, E2, W2):
    return pl.pallas_call(
        _proj_body,
        grid=(_NBLK,),
        in_specs=[
            pl.BlockSpec((_BLK, 128), lambda i: (jnp.minimum(i, _SEG1 - 1), 0)),
            pl.BlockSpec((128, _EMBED), lambda i: (0, 0)),
            pl.BlockSpec((_BLK, 32), lambda i: (jnp.clip(i - _SEG1, 0, 9), 0)),
            pl.BlockSpec((32, _EMBED), lambda i: (0, 0)),
            pl.BlockSpec((_BLK, 8), lambda i: (jnp.clip(i - _SEG2, 0, 234), 0)),
            pl.BlockSpec((8, _EMBED), lambda i: (0, 0)),
        ],
        out_specs=pl.BlockSpec((_BLK, _EMBED), lambda i: (i, 0)),
        out_shape=jax.ShapeDtypeStruct((_V_TOTAL, _EMBED), jnp.float32),
    )(E0, W0, E1, W1, E2, W2)


def _sc_gather(table, ids2d, n_tokens):
    info = plsc.get_sparse_core_info()
    nc, ns = info.num_cores, info.num_subcores
    nw = nc * ns
    n_groups = n_tokens // _LANES
    g_per_w = n_groups // nw
    n_iter = g_per_w // _CH_G
    ch_tokens = _CH_G * _LANES

    mesh = plsc.VectorSubcoreMesh(core_axis_name="c", subcore_axis_name="s")

    @functools.partial(
        pl.kernel,
        out_type=jax.ShapeDtypeStruct((n_tokens, _EMBED), jnp.float32),
        mesh=mesh,
        scratch_types=[
            pltpu.VMEM((_CH_G, _LANES), jnp.int32),
            pltpu.VMEM((ch_tokens, _EMBED), jnp.float32),
            pltpu.SemaphoreType.DMA,
        ],
    )
    def k(table_hbm, ids_hbm, out_hbm, idx_v, rows_v, sem):
        wid = lax.axis_index("s") * nc + lax.axis_index("c")
        g0 = wid * g_per_w

        def body(it, _):
            grow = g0 + it * _CH_G
            pltpu.sync_copy(ids_hbm.at[pl.ds(grow, _CH_G)], idx_v)
            cps = [
                pltpu.async_copy(
                    table_hbm.at[idx_v.at[j]],
                    rows_v.at[pl.ds(j * _LANES, _LANES)],
                    sem,
                )
                for j in range(_CH_G)
            ]
            for cp in cps:
                cp.wait()
            pltpu.sync_copy(
                rows_v, out_hbm.at[pl.ds(grow * _LANES, ch_tokens)]
            )
            return 0

        lax.fori_loop(0, n_iter, body, 0)

    return k(table, ids2d)


def kernel(input, E0, W0, E1, W1, E2, W2):
    table = _build_projected_table(E0, W0, E1, W1, E2, W2)
    b, s = input.shape
    n_tokens = b * s
    ids2d = input.reshape(n_tokens // _LANES, _LANES)
    out = _sc_gather(table, ids2d, n_tokens)
    return out.reshape(b, s, _EMBED)


# speedup vs baseline: 8.4997x; 1.0522x over previous
import functools

import jax
import jax.numpy as jnp
from jax import lax
from jax.experimental import pallas as pl
from jax.experimental.pallas import tpu as pltpu
from jax.experimental.pallas import tpu_sc as plsc

_EMBED = 128
_CUT0 = 20000
_CUT1 = 60000
_V_TOTAL = 1_000_000
_BLK = 10000
_NBLK = _V_TOTAL // _BLK
_SEG1 = _CUT0 // _BLK
_SEG2 = _CUT1 // _BLK

_LANES = 128
_CH_G = 4


def _proj_body(e0, w0, e1, w1, e2, w2, out):
    i = pl.program_id(0)

    @pl.when(i < _SEG1)
    def _():
        out[...] = jnp.dot(e0[...], w0[...], preferred_element_type=jnp.float32)

    @pl.when(jnp.logical_and(i >= _SEG1, i < _SEG2))
    def _():
        out[...] = jnp.dot(e1[...], w1[...], preferred_element_type=jnp.float32)

    @pl.when(i >= _SEG2)
    def _():
        out[...] = jnp.dot(e2[...], w2[...], preferred_element_type=jnp.float32)


def _build_projected_table(E0, W0, E1, W1, E2, W2):
    return pl.pallas_call(
        _proj_body,
        grid=(_NBLK,),
        in_specs=[
            pl.BlockSpec((_BLK, 128), lambda i: (jnp.minimum(i, _SEG1 - 1), 0)),
            pl.BlockSpec((128, _EMBED), lambda i: (0, 0)),
            pl.BlockSpec((_BLK, 32), lambda i: (jnp.clip(i - _SEG1, 0, 40000 // _BLK - 1), 0)),
            pl.BlockSpec((32, _EMBED), lambda i: (0, 0)),
            pl.BlockSpec((_BLK, 8), lambda i: (jnp.clip(i - _SEG2, 0, 940000 // _BLK - 1), 0)),
            pl.BlockSpec((8, _EMBED), lambda i: (0, 0)),
        ],
        out_specs=pl.BlockSpec((_BLK, _EMBED), lambda i: (i, 0)),
        out_shape=jax.ShapeDtypeStruct((_V_TOTAL, _EMBED), jnp.float32),
    )(E0, W0, E1, W1, E2, W2)


def _sc_gather(table, ids2d, n_tokens):
    info = plsc.get_sparse_core_info()
    nc, ns = info.num_cores, info.num_subcores
    nw = nc * ns
    n_groups = n_tokens // _LANES
    g_per_w = n_groups // nw
    n_iter = g_per_w // _CH_G
    ch_tokens = _CH_G * _LANES

    mesh = plsc.VectorSubcoreMesh(core_axis_name="c", subcore_axis_name="s")

    @functools.partial(
        pl.kernel,
        out_type=jax.ShapeDtypeStruct((n_tokens, _EMBED), jnp.float32),
        mesh=mesh,
        scratch_types=[
            pltpu.VMEM((_CH_G, _LANES), jnp.int32),
            pltpu.VMEM((ch_tokens, _EMBED), jnp.float32),
            pltpu.SemaphoreType.DMA,
        ],
    )
    def k(table_hbm, ids_hbm, out_hbm, idx_v, rows_v, sem):
        wid = lax.axis_index("s") * nc + lax.axis_index("c")
        g0 = wid * g_per_w

        def body(it, _):
            grow = g0 + it * _CH_G
            pltpu.sync_copy(ids_hbm.at[pl.ds(grow, _CH_G)], idx_v)
            cps = [
                pltpu.async_copy(
                    table_hbm.at[idx_v.at[j]],
                    rows_v.at[pl.ds(j * _LANES, _LANES)],
                    sem,
                )
                for j in range(_CH_G)
            ]
            for cp in cps:
                cp.wait()
            pltpu.sync_copy(
                rows_v, out_hbm.at[pl.ds(grow * _LANES, ch_tokens)]
            )
            return 0

        lax.fori_loop(0, n_iter, body, 0)

    return k(table, ids2d)


def kernel(input, E0, W0, E1, W1, E2, W2):
    table = _build_projected_table(E0, W0, E1, W1, E2, W2)
    b, s = input.shape
    n_tokens = b * s
    ids2d = input.reshape(n_tokens // _LANES, _LANES)
    out = _sc_gather(table, ids2d, n_tokens)
    return out.reshape(b, s, _EMBED)
